# Initial kernel scaffold; baseline (speedup 1.0000x reference)
import jax
import jax.numpy as jnp
from jax.experimental import pallas as pl


def kernel(formula_vectors, atom_table, count_table, pos_table, ln_weight, ln_bias):
    raise NotImplementedError("write your pallas kernel here")


# speedup vs baseline: 6.6196x; 6.6196x over previous
import jax
import jax.numpy as jnp
from jax.experimental import pallas as pl

_A = 30
_D = 128
_CMAX = 200
_CPAD = 208


def _body(fv_ref, atom_ref, cnt_ref, pos_ref, w_ref, b_ref, out_ref, mask_ref):
    fv = fv_ref[...]
    mask_ref[...] = (fv > 0).astype(jnp.float32)
    base = atom_ref[...] + pos_ref[...]
    cnt = cnt_ref[...]
    w = w_ref[...]
    b = b_ref[...]
    idx = jnp.clip(fv, 0, _CMAX)
    bb = fv.shape[0]
    for a in range(_A):
        col = idx[:, a:a + 1]
        oh = (jax.lax.broadcasted_iota(jnp.int32, (bb, _CPAD), 1) == col)
        oh = oh.astype(jnp.float32)
        emb = jnp.dot(oh, cnt, preferred_element_type=jnp.float32)
        x = emb + base[a:a + 1, :]
        mean = jnp.mean(x, axis=1, keepdims=True)
        xc = x - mean
        var = jnp.mean(xc * xc, axis=1, keepdims=True)
        y = xc * jax.lax.rsqrt(var + 1e-5) * w + b
        out_ref[:, a, :] = y


def kernel(formula_vectors, atom_table, count_table, pos_table, ln_weight, ln_bias):
    B, A = formula_vectors.shape
    D = atom_table.shape[1]
    cnt_pad = jnp.pad(count_table, ((0, _CPAD - count_table.shape[0]), (0, 0)))
    BB = 256
    grid = B // BB
    out, mask = pl.pallas_call(
        _body,
        grid=(grid,),
        in_specs=[
            pl.BlockSpec((BB, A), lambda i: (i, 0)),
            pl.BlockSpec((A, D), lambda i: (0, 0)),
            pl.BlockSpec((_CPAD, D), lambda i: (0, 0)),
            pl.BlockSpec((A, D), lambda i: (0, 0)),
            pl.BlockSpec((1, D), lambda i: (0, 0)),
            pl.BlockSpec((1, D), lambda i: (0, 0)),
        ],
        out_specs=[
            pl.BlockSpec((BB, A, D), lambda i: (i, 0, 0)),
            pl.BlockSpec((BB, A), lambda i: (i, 0)),
        ],
        out_shape=[
            jax.ShapeDtypeStruct((B, A, D), jnp.float32),
            jax.ShapeDtypeStruct((B, A), jnp.float32),
        ],
    )(formula_vectors, atom_table, cnt_pad, pos_table,
      ln_weight.reshape(1, D), ln_bias.reshape(1, D))
    return out, mask


# speedup vs baseline: 15.9607x; 2.4111x over previous
import jax
import jax.numpy as jnp
from jax import lax
from jax.experimental import pallas as pl
from jax.experimental.pallas import tpu as pltpu
from jax.experimental.pallas import tpu_sc as plsc

_A = 30
_D = 128
_CMAX = 200
_CPAD = 208


def _prep_body(fv_ref, atom_ref, cnt_ref, pos_ref, w_ref, b_ref,
               combo_ref, mask_ref):
    mask_ref[...] = (fv_ref[...] > 0).astype(jnp.float32)
    base = atom_ref[...] + pos_ref[...]
    x = base[:, None, :] + cnt_ref[...][None, :, :]
    mean = jnp.mean(x, axis=-1, keepdims=True)
    xc = x - mean
    var = jnp.mean(xc * xc, axis=-1, keepdims=True)
    combo_ref[...] = xc * lax.rsqrt(var + 1e-5) * w_ref[...][None] + b_ref[...][None]


def _sc_gather(combo, fv_lin, n_total):
    info = plsc.get_sparse_core_info()
    NC, NS = info.num_cores, info.num_subcores
    NW = NC * NS
    per_w = n_total // NW
    bpw = per_w // _A
    PH = 8
    bpp = bpw // PH
    CH = 120
    NCH = per_w // CH
    mesh = plsc.VectorSubcoreMesh(core_axis_name="c", subcore_axis_name="s")

    def body(combo_hbm, fv_hbm, out_hbm,
             combo_sh, fv_v, idx_v, rows0, rows1, gsem0, gsem1, wsem0, wsem1):
        cid = lax.axis_index("c")
        sid = lax.axis_index("s")
        wid = sid * NC + cid
        base = wid * per_w

        @pl.when(sid == 0)
        def _():
            pltpu.sync_copy(combo_hbm, combo_sh)

        iota = lax.iota(jnp.int32, 16)
        a0 = iota
        a1 = iota + 16
        m1 = iota < (_A - 16)

        def phase(h, _):
            pltpu.sync_copy(
                fv_hbm.at[pl.ds((wid * bpw + h * bpp) * _D, bpp * _D)], fv_v)

            def row(r, _):
                s0 = fv_v[pl.ds(r * _D, 16)]
                s1 = fv_v[pl.ds(r * _D + 16, 16)]
                i0 = a0 * _CPAD + lax.min(lax.max(s0, 0), _CMAX)
                i1 = a1 * _CPAD + lax.min(lax.max(s1, 0), _CMAX)
                dst = (h * bpp + r) * _A + iota
                plsc.store_scatter(idx_v, [dst], i0)
                plsc.store_scatter(idx_v, [dst + 16], i1, mask=m1)
                return 0

            return lax.fori_loop(0, bpp, row, 0)

        lax.fori_loop(0, PH, phase, 0)
        plsc.subcore_barrier()

        def gstart(c, rows, sem):
            return pltpu.async_copy(
                combo_sh.at[idx_v.at[pl.ds(c * CH, CH)]], rows, sem)

        bCH = CH // _A

        def wstart(c, rows, sem):
            b0 = wid * bpw + c * bCH
            for k in range(bCH):
                pltpu.async_copy(
                    rows.at[pl.ds(k * _A, _A)], out_hbm.at[b0 + k], sem)

        def wdrain(rows, sem):
            for k in range(bCH):
                pltpu.make_async_copy(
                    rows.at[pl.ds(k * _A, _A)], out_hbm.at[0], sem).wait()

        def pair(p, _):
            c0 = 2 * p
            c1 = c0 + 1

            @pl.when(p > 0)
            def _():
                wdrain(rows0, wsem0)
                wdrain(rows1, wsem1)

            g0 = gstart(c0, rows0, gsem0)
            g1 = gstart(c1, rows1, gsem1)
            g0.wait()
            wstart(c0, rows0, wsem0)
            g1.wait()
            wstart(c1, rows1, wsem1)
            return 0

        lax.fori_loop(0, NCH // 2, pair, 0)
        wdrain(rows0, wsem0)
        wdrain(rows1, wsem1)

    out = pl.kernel(
        body,
        out_type=jax.ShapeDtypeStruct((n_total // _A, _A, _D), jnp.float32),
        mesh=mesh,
        compiler_params=pltpu.CompilerParams(
            needs_layout_passes=False, use_tc_tiling_on_sc=True),
        scratch_types=[
            pltpu.VMEM_SHARED((_A * _CPAD, _D), jnp.float32),
            pltpu.VMEM((bpp * _D,), jnp.int32),
            pltpu.VMEM((per_w,), jnp.int32),
            pltpu.VMEM((CH, _D), jnp.float32),
            pltpu.VMEM((CH, _D), jnp.float32),
            pltpu.SemaphoreType.DMA,
            pltpu.SemaphoreType.DMA,
            pltpu.SemaphoreType.DMA,
            pltpu.SemaphoreType.DMA,
        ],
    )(combo, fv_lin)
    return out


def kernel(formula_vectors, atom_table, count_table, pos_table, ln_weight, ln_bias):
    B, A = formula_vectors.shape
    D = atom_table.shape[1]
    cnt_pad = jnp.pad(count_table, ((0, _CPAD - count_table.shape[0]), (0, 0)))
    combo3, mask = pl.pallas_call(
        _prep_body,
        out_shape=[
            jax.ShapeDtypeStruct((_A, _CPAD, D), jnp.float32),
            jax.ShapeDtypeStruct((B, A), jnp.float32),
        ],
    )(formula_vectors, atom_table, cnt_pad, pos_table,
      ln_weight.reshape(1, D), ln_bias.reshape(1, D))
    combo = combo3.reshape(_A * _CPAD, D)
    fv_lin = jnp.pad(formula_vectors, ((0, 0), (0, _D - A))).reshape(B * _D)
    out = _sc_gather(combo, fv_lin, B * A)
    return out, mask


# speedup vs baseline: 35.6091x; 2.2310x over previous
import jax
import jax.numpy as jnp
from jax import lax
from jax.experimental import pallas as pl
from jax.experimental.pallas import tpu as pltpu
from jax.experimental.pallas import tpu_sc as plsc

_A = 30
_D = 128
_CMAX = 200
_CPAD = 208


def _prep_body(fvt_ref, atom_ref, cnt_ref, pos_ref, w_ref, b_ref,
               combo_ref, maskt_ref):
    maskt_ref[...] = (fvt_ref[...] > 0).astype(jnp.float32)
    base = atom_ref[...] + pos_ref[...]
    x = base[:, None, :] + cnt_ref[...][None, :, :]
    mean = jnp.mean(x, axis=-1, keepdims=True)
    xc = x - mean
    var = jnp.mean(xc * xc, axis=-1, keepdims=True)
    combo_ref[...] = xc * lax.rsqrt(var + 1e-5) * w_ref[...][None] + b_ref[...][None]


def _sc_gather(combo, fvt_lin, n_total, B):
    info = plsc.get_sparse_core_info()
    NC, NS = info.num_cores, info.num_subcores
    NW = NC * NS
    per_w = n_total // NW
    CH = 240
    NCH = per_w // CH
    shift = B.bit_length() - 1
    mesh = plsc.VectorSubcoreMesh(core_axis_name="c", subcore_axis_name="s")

    def body(combo_hbm, fv_hbm, out_hbm,
             combo_sh, idx_v, rows0, rows1, gsem0, gsem1, wsem0, wsem1):
        cid = lax.axis_index("c")
        sid = lax.axis_index("s")
        wid = sid * NC + cid
        q0 = wid * per_w

        @pl.when(sid == 0)
        def _():
            pltpu.sync_copy(combo_hbm, combo_sh)

        pltpu.sync_copy(fv_hbm.at[pl.ds(q0, per_w)], idx_v)

        def idx_body(j, _):
            v16 = idx_v[pl.ds(j * 16, 16)]
            a = lax.shift_right_logical(q0 + j * 16, shift)
            idx_v[pl.ds(j * 16, 16)] = a * _CPAD + lax.min(lax.max(v16, 0), _CMAX)
            return 0

        lax.fori_loop(0, per_w // 16, idx_body, 0)
        plsc.subcore_barrier()

        def gstart(c, rows, sem):
            return pltpu.async_copy(
                combo_sh.at[idx_v.at[pl.ds(c * CH, CH)]], rows, sem)

        def wstart(c, rows, sem):
            pltpu.async_copy(rows, out_hbm.at[pl.ds(q0 + c * CH, CH)], sem)

        def wdrain(rows, sem):
            pltpu.make_async_copy(rows, out_hbm.at[pl.ds(q0, CH)], sem).wait()

        def pair(p, _):
            c0 = 2 * p
            c1 = c0 + 1

            @pl.when(p > 0)
            def _():
                wdrain(rows0, wsem0)
                wdrain(rows1, wsem1)

            g0 = gstart(c0, rows0, gsem0)
            g1 = gstart(c1, rows1, gsem1)
            g0.wait()
            wstart(c0, rows0, wsem0)
            g1.wait()
            wstart(c1, rows1, wsem1)
            return 0

        lax.fori_loop(0, NCH // 2, pair, 0)
        wdrain(rows0, wsem0)
        wdrain(rows1, wsem1)

    out = pl.kernel(
        body,
        out_type=jax.ShapeDtypeStruct((n_total, _D), jnp.float32),
        mesh=mesh,
        compiler_params=pltpu.CompilerParams(
            needs_layout_passes=False, use_tc_tiling_on_sc=True),
        scratch_types=[
            pltpu.VMEM_SHARED((_A * _CPAD, _D), jnp.float32),
            pltpu.VMEM((per_w,), jnp.int32),
            pltpu.VMEM((CH, _D), jnp.float32),
            pltpu.VMEM((CH, _D), jnp.float32),
            pltpu.SemaphoreType.DMA,
            pltpu.SemaphoreType.DMA,
            pltpu.SemaphoreType.DMA,
            pltpu.SemaphoreType.DMA,
        ],
    )(combo, fvt_lin)
    return out


def kernel(formula_vectors, atom_table, count_table, pos_table, ln_weight, ln_bias):
    B, A = formula_vectors.shape
    D = atom_table.shape[1]
    cnt_pad = jnp.pad(count_table, ((0, _CPAD - count_table.shape[0]), (0, 0)))
    fvt = formula_vectors.T
    combo3, maskt = pl.pallas_call(
        _prep_body,
        out_shape=[
            jax.ShapeDtypeStruct((_A, _CPAD, D), jnp.float32),
            jax.ShapeDtypeStruct((A, B), jnp.float32),
        ],
    )(fvt, atom_table, cnt_pad, pos_table,
      ln_weight.reshape(1, D), ln_bias.reshape(1, D))
    fvt_lin = jnp.pad(fvt, ((0, 2), (0, 0))).reshape(32 * B)
    out = _sc_gather(combo3.reshape(_A * _CPAD, D), fvt_lin, B * A, B)
    return out.reshape(A, B, D).transpose(1, 0, 2), maskt.T


# speedup vs baseline: 49.0971x; 1.3788x over previous
import jax
import jax.numpy as jnp
from jax import lax
from jax.experimental import pallas as pl
from jax.experimental.pallas import tpu as pltpu
from jax.experimental.pallas import tpu_sc as plsc

_A = 30
_D = 128
_CMAX = 200
_CPAD = 208


def _prep_body(fvt_ref, atom_ref, cnt_ref, pos_ref, w_ref, b_ref,
               combo_ref, maskt_ref):
    maskt_ref[...] = (fvt_ref[...] > 0).astype(jnp.float32)
    base = atom_ref[...] + pos_ref[...]
    x = base[:, None, :] + cnt_ref[...][None, :, :]
    mean = jnp.mean(x, axis=-1, keepdims=True)
    xc = x - mean
    var = jnp.mean(xc * xc, axis=-1, keepdims=True)
    combo_ref[...] = xc * lax.rsqrt(var + 1e-5) * w_ref[...][None] + b_ref[...][None]


def _sc_gather(combo, fvt_lin, n_total, B):
    info = plsc.get_sparse_core_info()
    NC, NS = info.num_cores, info.num_subcores
    NW = NC * NS
    per_w = n_total // NW
    CH = 120
    NCH = per_w // CH
    shift = B.bit_length() - 1
    mesh = plsc.VectorSubcoreMesh(core_axis_name="c", subcore_axis_name="s")

    def body(combo_hbm, fv_hbm, out_hbm,
             combo_sh, idx_v, rows0, rows1, rows2, rows3,
             gsem0, gsem1, gsem2, gsem3, wsem0, wsem1, wsem2, wsem3):
        rows_bufs = (rows0, rows1, rows2, rows3)
        gsems = (gsem0, gsem1, gsem2, gsem3)
        wsems = (wsem0, wsem1, wsem2, wsem3)
        cid = lax.axis_index("c")
        sid = lax.axis_index("s")
        wid = sid * NC + cid
        q0 = wid * per_w

        @pl.when(sid == 0)
        def _():
            pltpu.sync_copy(combo_hbm, combo_sh)

        pltpu.sync_copy(fv_hbm.at[pl.ds(q0, per_w)], idx_v)

        def idx_body(j, _):
            v16 = idx_v[pl.ds(j * 16, 16)]
            a = lax.shift_right_logical(q0 + j * 16, shift)
            idx_v[pl.ds(j * 16, 16)] = a * _CPAD + lax.min(lax.max(v16, 0), _CMAX)
            return 0

        lax.fori_loop(0, per_w // 16, idx_body, 0)
        plsc.subcore_barrier()

        def gstart(c, rows, sem):
            return pltpu.async_copy(
                combo_sh.at[idx_v.at[pl.ds(c * CH, CH)]], rows, sem)

        def wstart(c, rows, sem):
            pltpu.async_copy(rows, out_hbm.at[pl.ds(q0 + c * CH, CH)], sem)

        def wdrain(rows, sem):
            pltpu.make_async_copy(rows, out_hbm.at[pl.ds(q0, CH)], sem).wait()

        def quad(p, _):
            handles = []
            for k in range(4):
                @pl.when(p > 0)
                def _(k=k):
                    wdrain(rows_bufs[k], wsems[k])
                handles.append(gstart(4 * p + k, rows_bufs[k], gsems[k]))
            for k in range(4):
                handles[k].wait()
                wstart(4 * p + k, rows_bufs[k], wsems[k])
            return 0

        lax.fori_loop(0, NCH // 4, quad, 0)
        for k in range(4):
            wdrain(rows_bufs[k], wsems[k])

    out = pl.kernel(
        body,
        out_type=jax.ShapeDtypeStruct((n_total, _D), jnp.float32),
        mesh=mesh,
        compiler_params=pltpu.CompilerParams(
            needs_layout_passes=False, use_tc_tiling_on_sc=True),
        scratch_types=[
            pltpu.VMEM_SHARED((_A * _CPAD, _D), jnp.float32),
            pltpu.VMEM((per_w,), jnp.int32),
            pltpu.VMEM((CH, _D), jnp.float32),
            pltpu.VMEM((CH, _D), jnp.float32),
            pltpu.VMEM((CH, _D), jnp.float32),
            pltpu.VMEM((CH, _D), jnp.float32),
            pltpu.SemaphoreType.DMA,
            pltpu.SemaphoreType.DMA,
            pltpu.SemaphoreType.DMA,
            pltpu.SemaphoreType.DMA,
            pltpu.SemaphoreType.DMA,
            pltpu.SemaphoreType.DMA,
            pltpu.SemaphoreType.DMA,
            pltpu.SemaphoreType.DMA,
        ],
    )(combo, fvt_lin)
    return out


def kernel(formula_vectors, atom_table, count_table, pos_table, ln_weight, ln_bias):
    B, A = formula_vectors.shape
    D = atom_table.shape[1]
    cnt_pad = jnp.pad(count_table, ((0, _CPAD - count_table.shape[0]), (0, 0)))
    fvt = formula_vectors.T
    combo3, maskt = pl.pallas_call(
        _prep_body,
        out_shape=[
            jax.ShapeDtypeStruct((_A, _CPAD, D), jnp.float32),
            jax.ShapeDtypeStruct((A, B), jnp.float32),
        ],
    )(fvt, atom_table, cnt_pad, pos_table,
      ln_weight.reshape(1, D), ln_bias.reshape(1, D))
    fvt_lin = jnp.pad(fvt, ((0, 2), (0, 0))).reshape(32 * B)
    out = _sc_gather(combo3.reshape(_A * _CPAD, D), fvt_lin, B * A, B)
    return out.reshape(A, B, D).transpose(1, 0, 2), maskt.T


# speedup vs baseline: 49.1776x; 1.0016x over previous
import jax
import jax.numpy as jnp
from jax import lax
from jax.experimental import pallas as pl
from jax.experimental.pallas import tpu as pltpu
from jax.experimental.pallas import tpu_sc as plsc

_A = 30
_D = 128
_CMAX = 200
_CPAD = 208


def _prep_body(fvt_ref, atom_ref, cnt_ref, pos_ref, w_ref, b_ref,
               combo_ref, maskt_ref):
    maskt_ref[...] = (fvt_ref[...] > 0).astype(jnp.float32)
    base = atom_ref[...] + pos_ref[...]
    x = base[:, None, :] + cnt_ref[...][None, :, :]
    mean = jnp.mean(x, axis=-1, keepdims=True)
    xc = x - mean
    var = jnp.mean(xc * xc, axis=-1, keepdims=True)
    combo_ref[...] = xc * lax.rsqrt(var + 1e-5) * w_ref[...][None] + b_ref[...][None]


def _sc_gather(combo, fvt_lin, n_total, B):
    info = plsc.get_sparse_core_info()
    NC, NS = info.num_cores, info.num_subcores
    NW = NC * NS
    per_w = n_total // NW
    CH = 80
    NB = 6
    NCH = per_w // CH
    shift = B.bit_length() - 1
    mesh = plsc.VectorSubcoreMesh(core_axis_name="c", subcore_axis_name="s")

    def body(combo_hbm, fv_hbm, out_hbm, combo_sh, idx_v, *bufs):
        rows_bufs = bufs[:NB]
        gsems = bufs[NB:2 * NB]
        wsems = bufs[2 * NB:]
        cid = lax.axis_index("c")
        sid = lax.axis_index("s")
        wid = sid * NC + cid
        q0 = wid * per_w

        @pl.when(sid == 0)
        def _():
            pltpu.sync_copy(combo_hbm, combo_sh)

        pltpu.sync_copy(fv_hbm.at[pl.ds(q0, per_w)], idx_v)

        def idx_body(j, _):
            v16 = idx_v[pl.ds(j * 16, 16)]
            a = lax.shift_right_logical(q0 + j * 16, shift)
            idx_v[pl.ds(j * 16, 16)] = a * _CPAD + lax.min(lax.max(v16, 0), _CMAX)
            return 0

        lax.fori_loop(0, per_w // 16, idx_body, 0)
        plsc.subcore_barrier()

        def gstart(c, rows, sem):
            return pltpu.async_copy(
                combo_sh.at[idx_v.at[pl.ds(c * CH, CH)]], rows, sem)

        def wstart(c, rows, sem):
            pltpu.async_copy(rows, out_hbm.at[pl.ds(q0 + c * CH, CH)], sem)

        def wdrain(rows, sem):
            pltpu.make_async_copy(rows, out_hbm.at[pl.ds(q0, CH)], sem).wait()

        def ring(p, _):
            handles = []
            for k in range(NB):
                @pl.when(p > 0)
                def _(k=k):
                    wdrain(rows_bufs[k], wsems[k])
                handles.append(gstart(NB * p + k, rows_bufs[k], gsems[k]))
            for k in range(NB):
                handles[k].wait()
                wstart(NB * p + k, rows_bufs[k], wsems[k])
            return 0

        lax.fori_loop(0, NCH // NB, ring, 0)
        for k in range(NB):
            wdrain(rows_bufs[k], wsems[k])

    out = pl.kernel(
        body,
        out_type=jax.ShapeDtypeStruct((n_total, _D), jnp.float32),
        mesh=mesh,
        compiler_params=pltpu.CompilerParams(
            needs_layout_passes=False, use_tc_tiling_on_sc=True),
        scratch_types=[
            pltpu.VMEM_SHARED((_A * _CPAD, _D), jnp.float32),
            pltpu.VMEM((per_w,), jnp.int32),
            *([pltpu.VMEM((CH, _D), jnp.float32)] * NB),
            *([pltpu.SemaphoreType.DMA] * (2 * NB)),
        ],
    )(combo, fvt_lin)
    return out


def kernel(formula_vectors, atom_table, count_table, pos_table, ln_weight, ln_bias):
    B, A = formula_vectors.shape
    D = atom_table.shape[1]
    cnt_pad = jnp.pad(count_table, ((0, _CPAD - count_table.shape[0]), (0, 0)))
    fvt = formula_vectors.T
    combo3, maskt = pl.pallas_call(
        _prep_body,
        out_shape=[
            jax.ShapeDtypeStruct((_A, _CPAD, D), jnp.float32),
            jax.ShapeDtypeStruct((A, B), jnp.float32),
        ],
    )(fvt, atom_table, cnt_pad, pos_table,
      ln_weight.reshape(1, D), ln_bias.reshape(1, D))
    fvt_lin = jnp.pad(fvt, ((0, 2), (0, 0))).reshape(32 * B)
    out = _sc_gather(combo3.reshape(_A * _CPAD, D), fvt_lin, B * A, B)
    return out.reshape(A, B, D).transpose(1, 0, 2), maskt.T


# speedup vs baseline: 50.7487x; 1.0319x over previous
import jax
import jax.numpy as jnp
from jax import lax
from jax.experimental import pallas as pl
from jax.experimental.pallas import tpu as pltpu
from jax.experimental.pallas import tpu_sc as plsc

_A = 30
_D = 128
_CMAX = 200
_CPAD = 208


def _prep_body(fvt_ref, atom_ref, cnt_ref, pos_ref, w_ref, b_ref,
               combo_ref, maskt_ref, fvp_ref):
    fvt = fvt_ref[...]
    maskt_ref[...] = (fvt > 0).astype(jnp.float32)
    fvp_ref[0:_A, :] = fvt
    fvp_ref[_A:, :] = jnp.zeros(
        (fvp_ref.shape[0] - _A, fvt.shape[1]), jnp.int32)
    base = atom_ref[...] + pos_ref[...]
    x = base[:, None, :] + cnt_ref[...][None, :, :]
    mean = jnp.mean(x, axis=-1, keepdims=True)
    xc = x - mean
    var = jnp.mean(xc * xc, axis=-1, keepdims=True)
    combo_ref[...] = xc * lax.rsqrt(var + 1e-5) * w_ref[...][None] + b_ref[...][None]


def _sc_gather(combo, fvt_lin, n_total, B):
    info = plsc.get_sparse_core_info()
    NC, NS = info.num_cores, info.num_subcores
    NW = NC * NS
    per_w = n_total // NW
    CH = 80
    NB = 6
    NCH = per_w // CH
    shift = B.bit_length() - 1
    mesh = plsc.VectorSubcoreMesh(core_axis_name="c", subcore_axis_name="s")

    def body(combo_hbm, fv_hbm, out_hbm, combo_sh, idx_v, *bufs):
        rows_bufs = bufs[:NB]
        gsems = bufs[NB:2 * NB]
        wsems = bufs[2 * NB:]
        cid = lax.axis_index("c")
        sid = lax.axis_index("s")
        wid = sid * NC + cid
        q0 = wid * per_w

        stg = (_A * _CPAD) // 15
        @pl.when(sid < 15)
        def _():
            pltpu.sync_copy(combo_hbm.at[pl.ds(sid * stg, stg)],
                            combo_sh.at[pl.ds(sid * stg, stg)])

        pltpu.sync_copy(fv_hbm.at[pl.ds(q0, per_w)], idx_v)

        def idx_body(j, _):
            v16 = idx_v[pl.ds(j * 16, 16)]
            a = lax.shift_right_logical(q0 + j * 16, shift)
            idx_v[pl.ds(j * 16, 16)] = a * _CPAD + lax.min(lax.max(v16, 0), _CMAX)
            return 0

        spr = (NB * CH) // 16
        lax.fori_loop(0, spr, idx_body, 0)
        plsc.subcore_barrier()

        def gstart(c, rows, sem):
            return pltpu.async_copy(
                combo_sh.at[idx_v.at[pl.ds(c * CH, CH)]], rows, sem)

        def wstart(c, rows, sem):
            pltpu.async_copy(rows, out_hbm.at[pl.ds(q0 + c * CH, CH)], sem)

        def wdrain(rows, sem):
            pltpu.make_async_copy(rows, out_hbm.at[pl.ds(q0, CH)], sem).wait()

        def ring(p, _):
            handles = []
            for k in range(NB):
                @pl.when(p > 0)
                def _(k=k):
                    wdrain(rows_bufs[k], wsems[k])
                handles.append(gstart(NB * p + k, rows_bufs[k], gsems[k]))
            @pl.when(p + 1 < NCH // NB)
            def _():
                lax.fori_loop((p + 1) * spr, (p + 2) * spr, idx_body, 0)
            for k in range(NB):
                handles[k].wait()
                wstart(NB * p + k, rows_bufs[k], wsems[k])
            return 0

        lax.fori_loop(0, NCH // NB, ring, 0)
        for k in range(NB):
            wdrain(rows_bufs[k], wsems[k])

    out = pl.kernel(
        body,
        out_type=jax.ShapeDtypeStruct((n_total, _D), jnp.float32),
        mesh=mesh,
        compiler_params=pltpu.CompilerParams(
            needs_layout_passes=False, use_tc_tiling_on_sc=True),
        scratch_types=[
            pltpu.VMEM_SHARED((_A * _CPAD, _D), jnp.float32),
            pltpu.VMEM((per_w,), jnp.int32),
            *([pltpu.VMEM((CH, _D), jnp.float32)] * NB),
            *([pltpu.SemaphoreType.DMA] * (2 * NB)),
        ],
    )(combo, fvt_lin)
    return out


def kernel(formula_vectors, atom_table, count_table, pos_table, ln_weight, ln_bias):
    B, A = formula_vectors.shape
    D = atom_table.shape[1]
    cnt_pad = jnp.pad(count_table, ((0, _CPAD - count_table.shape[0]), (0, 0)))
    fvt = formula_vectors.T
    combo3, maskt, fvp = pl.pallas_call(
        _prep_body,
        out_shape=[
            jax.ShapeDtypeStruct((_A, _CPAD, D), jnp.float32),
            jax.ShapeDtypeStruct((A, B), jnp.float32),
            jax.ShapeDtypeStruct((32, B), jnp.int32),
        ],
    )(fvt, atom_table, cnt_pad, pos_table,
      ln_weight.reshape(1, D), ln_bias.reshape(1, D))
    out = _sc_gather(combo3.reshape(_A * _CPAD, D), fvp.reshape(32 * B), B * A, B)
    return out.reshape(A, B, D).transpose(1, 0, 2), maskt.T
